# baseline (speedup 1.0000x reference)
import functools

import jax
import jax.numpy as jnp
from jax import lax
from jax.experimental import pallas as pl
from jax.experimental.pallas import tpu as pltpu
from jax.experimental.pallas import tpu_sc as plsc

_H = 14
_W = 14
_S = _H * _W
_B = 64
_C = 768
_CG = 128
_L = 16
_NW = 32
_NCG = _C // _CG
_BH = _B // 2
_UNITS = _BH * _NCG
_UPW = _UNITS // _NW


def _unit_pos(uid, b_off):
    return b_off + uid // _NCG, (uid % _NCG) * _CG


def _in_copy(x_hbm, b, c0, buf, sem):
    return pltpu.make_async_copy(
        x_hbm.at[b, :, pl.ds(c0, _CG)], buf, sem)


def _out_copy(buf, dst_hbm, b, c0, sem):
    return pltpu.make_async_copy(
        buf, dst_hbm.at[b, :, pl.ds(c0, _CG)], sem)


def _sc_body(b_off, x_hbm, tp_hbm, masked_hbm, tmpl_hbm, tp_v, x_a, x_b,
             t_v, sem_xa, sem_xb, sem_t, sem_ma, sem_mb):
    wid = lax.axis_index("s") * 2 + lax.axis_index("c")
    pltpu.sync_copy(tp_hbm, tp_v)

    def compute(x2):
        for g in range(_CG // _L):
            gl = g * _L

            mx0 = jnp.full((_L,), -jnp.inf, jnp.float32)
            am0 = jnp.zeros((_L,), jnp.int32)

            @plsc.parallel_loop(0, _S, 4, unroll=7, carry=(mx0, am0))
            def amax_loop(s, carry):
                mx, am = carry
                v0 = x2[s, pl.ds(gl, _L)]
                v1 = x2[s + 1, pl.ds(gl, _L)]
                v2 = x2[s + 2, pl.ds(gl, _L)]
                v3 = x2[s + 3, pl.ds(gl, _L)]
                b1 = v1 > v0
                m01 = jnp.where(b1, v1, v0)
                a01 = jnp.where(b1, s + 1, s)
                b3 = v3 > v2
                m23 = jnp.where(b3, v3, v2)
                a23 = jnp.where(b3, s + 3, s + 2)
                bb = m23 > m01
                ml = jnp.where(bb, m23, m01)
                al = jnp.where(bb, a23, a01)
                bc = ml > mx
                return (jnp.where(bc, ml, mx), jnp.where(bc, al, am))

            _, am = amax_loop
            base = am * _S

            @plsc.parallel_loop(0, _S, 2, unroll=7)
            def out_loop(s):
                for d in range(2):
                    tv = plsc.load_gather(tp_v, [base + (s + d)])
                    xv = x2[s + d, pl.ds(gl, _L)]
                    t_v[s + d, pl.ds(gl, _L)] = tv
                    x2[s + d, pl.ds(gl, _L)] = jnp.maximum(xv * tv, 0.0)

    b0, c00 = _unit_pos(wid, b_off)
    _in_copy(x_hbm, b0, c00, t_v, sem_t).start()
    _in_copy(x_hbm, b0, c00, t_v, sem_mb).start()
    _in_copy(x_hbm, b0, c00, x_a, sem_xa).start()

    def pair_body(j, _):
        u0 = (2 * j) * _NW + wid
        u1 = (2 * j + 1) * _NW + wid
        un = jnp.minimum(2 * j + 2, _UPW - 1) * _NW + wid
        bu0, cu0 = _unit_pos(u0, b_off)
        bu1, cu1 = _unit_pos(u1, b_off)
        bun, cun = _unit_pos(un, b_off)
        ou0, _ = _unit_pos(u0, 0)
        ou1, _ = _unit_pos(u1, 0)

        _out_copy(x_b, masked_hbm, ou0, cu0, sem_mb).wait()
        _in_copy(x_hbm, bu1, cu1, x_b, sem_xb).start()
        _in_copy(x_hbm, bu0, cu0, x_a, sem_xa).wait()
        _out_copy(t_v, tmpl_hbm, ou0, cu0, sem_t).wait()
        compute(x_a)
        _out_copy(t_v, tmpl_hbm, ou0, cu0, sem_t).start()
        _out_copy(x_a, masked_hbm, ou0, cu0, sem_ma).start()

        _out_copy(x_a, masked_hbm, ou1, cu1, sem_ma).wait()
        _in_copy(x_hbm, bun, cun, x_a, sem_xa).start()
        _in_copy(x_hbm, bu1, cu1, x_b, sem_xb).wait()
        _out_copy(t_v, tmpl_hbm, ou1, cu1, sem_t).wait()
        compute(x_b)
        _out_copy(t_v, tmpl_hbm, ou1, cu1, sem_t).start()
        _out_copy(x_b, masked_hbm, ou1, cu1, sem_mb).start()
        return 0

    lax.fori_loop(0, _UPW // 2, pair_body, 0)

    bl, cl = _unit_pos((_UPW - 1) * _NW + wid, b_off)
    ol, _ = _unit_pos((_UPW - 1) * _NW + wid, 0)
    _out_copy(t_v, tmpl_hbm, ol, cl, sem_t).wait()
    _out_copy(x_b, masked_hbm, ol, cl, sem_mb).wait()
    _in_copy(x_hbm, bl, cl, x_a, sem_xa).wait()


def kernel(input, t_p):
    x = input
    b, h, w, c = x.shape
    s = h * w

    mesh = plsc.VectorSubcoreMesh(core_axis_name="c", subcore_axis_name="s")

    def make_run(b_off):
        return functools.partial(
            pl.kernel,
            out_type=[
                jax.ShapeDtypeStruct((_BH, s, c), jnp.float32),
                jax.ShapeDtypeStruct((_BH, s, c), jnp.float32),
            ],
            mesh=mesh,
            compiler_params=pltpu.CompilerParams(needs_layout_passes=False),
            scratch_types=[
                pltpu.VMEM((_S * _S,), jnp.float32),
                pltpu.VMEM((_S, _CG), jnp.float32),
                pltpu.VMEM((_S, _CG), jnp.float32),
                pltpu.VMEM((_S, _CG), jnp.float32),
                pltpu.SemaphoreType.DMA,
                pltpu.SemaphoreType.DMA,
                pltpu.SemaphoreType.DMA,
                pltpu.SemaphoreType.DMA,
                pltpu.SemaphoreType.DMA,
            ],
        )(functools.partial(_sc_body, b_off))

    x3 = x.reshape(b, s, c)
    tp1 = t_p.reshape(s * s)
    masked0, tmpl0 = make_run(0)(x3, tp1)
    masked1, tmpl1 = make_run(_BH)(x3, tp1)
    masked = jnp.concatenate([masked0, masked1], axis=0)
    tmpl = jnp.concatenate([tmpl0, tmpl1], axis=0)
    return (masked.reshape(b, h, w, c), x, tmpl.reshape(b, h, w, c))


# speedup vs baseline: 1.2859x; 1.2859x over previous
import functools

import jax
import jax.numpy as jnp
from jax import lax
from jax.experimental import pallas as pl
from jax.experimental.pallas import tpu as pltpu
from jax.experimental.pallas import tpu_sc as plsc

_H = 14
_W = 14
_S = _H * _W
_B = 64
_C = 768
_CG = 128
_L = 16
_NW = 32
_NCG = _C // _CG
_UNITS = _B * _NCG
_UPW = _UNITS // _NW


def _unit_pos(uid):
    return uid // _NCG, (uid % _NCG) * _CG


def _in_copies(x_hbm, b, c0, buf, sem):
    return [
        pltpu.make_async_copy(
            x_hbm.at[b, :, pl.ds(c0, _CG)], buf, sem)
    ]


def _buf_out(buf, dst_hbm, b, c0, sem):
    return [
        pltpu.make_async_copy(
            buf, dst_hbm.at[b, :, pl.ds(c0, _CG)], sem)
    ]


def _sc_body(x_hbm, tp_hbm, masked_hbm, tmpl_hbm, tp_v, x_a, x_b, t_v,
             sem_xa, sem_xb, sem_t, sem_ma, sem_mb):
    wid = lax.axis_index("s") * 2 + lax.axis_index("c")
    pltpu.sync_copy(tp_hbm, tp_v)

    def compute(x2):
        for g in range(_CG // _L):
            gl = g * _L

            mx0 = jnp.full((_L,), -jnp.inf, jnp.float32)
            am0 = jnp.zeros((_L,), jnp.int32)

            @plsc.parallel_loop(0, _S, 4, unroll=7, carry=(mx0, am0))
            def amax_loop(s, carry):
                mx, am = carry
                v0 = x2[s, pl.ds(gl, _L)]
                v1 = x2[s + 1, pl.ds(gl, _L)]
                v2 = x2[s + 2, pl.ds(gl, _L)]
                v3 = x2[s + 3, pl.ds(gl, _L)]
                b1 = v1 > v0
                m01 = jnp.where(b1, v1, v0)
                a01 = jnp.where(b1, s + 1, s)
                b3 = v3 > v2
                m23 = jnp.where(b3, v3, v2)
                a23 = jnp.where(b3, s + 3, s + 2)
                bb = m23 > m01
                ml = jnp.where(bb, m23, m01)
                al = jnp.where(bb, a23, a01)
                bc = ml > mx
                return (jnp.where(bc, ml, mx), jnp.where(bc, al, am))

            _, am = amax_loop
            base = am * _S

            @plsc.parallel_loop(0, _S, 4, unroll=7)
            def out_loop(s):
                for d in range(4):
                    tv = plsc.load_gather(tp_v, [base + (s + d)])
                    xv = x2[s + d, pl.ds(gl, _L)]
                    t_v[s + d, pl.ds(gl, _L)] = tv
                    x2[s + d, pl.ds(gl, _L)] = jnp.maximum(xv * tv, 0.0)

    b0, c00 = _unit_pos(wid)
    for sem in (sem_t, sem_mb):
        for cp in _in_copies(x_hbm, b0, c00, t_v, sem):
            cp.start()
    for cp in _in_copies(x_hbm, b0, c00, x_a, sem_xa):
        cp.start()

    def pair_body(j, _):
        u0 = (2 * j) * _NW + wid
        u1 = (2 * j + 1) * _NW + wid
        un = jnp.minimum(2 * j + 2, _UPW - 1) * _NW + wid
        bu0, cu0 = _unit_pos(u0)
        bu1, cu1 = _unit_pos(u1)
        bun, cun = _unit_pos(un)

        for cp in _buf_out(x_b, masked_hbm, bu0, cu0, sem_mb):
            cp.wait()
        for cp in _in_copies(x_hbm, bu1, cu1, x_b, sem_xb):
            cp.start()
        for cp in _in_copies(x_hbm, bu0, cu0, x_a, sem_xa):
            cp.wait()
        for cp in _buf_out(t_v, tmpl_hbm, bu0, cu0, sem_t):
            cp.wait()
        compute(x_a)
        for cp in _buf_out(t_v, tmpl_hbm, bu0, cu0, sem_t):
            cp.start()
        for cp in _buf_out(x_a, masked_hbm, bu0, cu0, sem_ma):
            cp.start()

        for cp in _buf_out(x_a, masked_hbm, bu1, cu1, sem_ma):
            cp.wait()
        for cp in _in_copies(x_hbm, bun, cun, x_a, sem_xa):
            cp.start()
        for cp in _in_copies(x_hbm, bu1, cu1, x_b, sem_xb):
            cp.wait()
        for cp in _buf_out(t_v, tmpl_hbm, bu1, cu1, sem_t):
            cp.wait()
        compute(x_b)
        for cp in _buf_out(t_v, tmpl_hbm, bu1, cu1, sem_t):
            cp.start()
        for cp in _buf_out(x_b, masked_hbm, bu1, cu1, sem_mb):
            cp.start()
        return 0

    lax.fori_loop(0, _UPW // 2, pair_body, 0)

    bl, cl = _unit_pos((_UPW - 1) * _NW + wid)
    for cp in _buf_out(t_v, tmpl_hbm, bl, cl, sem_t):
        cp.wait()
    for cp in _buf_out(x_b, masked_hbm, bl, cl, sem_mb):
        cp.wait()
    for cp in _in_copies(x_hbm, bl, cl, x_a, sem_xa):
        cp.wait()


def kernel(input, t_p):
    x = input
    b, h, w, c = x.shape

    mesh = plsc.VectorSubcoreMesh(core_axis_name="c", subcore_axis_name="s")
    s = h * w
    run = functools.partial(
        pl.kernel,
        out_type=[
            jax.ShapeDtypeStruct((b, s, c), jnp.float32),
            jax.ShapeDtypeStruct((b, s, c), jnp.float32),
        ],
        mesh=mesh,
        compiler_params=pltpu.CompilerParams(needs_layout_passes=False),
        scratch_types=[
            pltpu.VMEM((_S * _S,), jnp.float32),
            pltpu.VMEM((_S, _CG), jnp.float32),
            pltpu.VMEM((_S, _CG), jnp.float32),
            pltpu.VMEM((_S, _CG), jnp.float32),
            pltpu.SemaphoreType.DMA,
            pltpu.SemaphoreType.DMA,
            pltpu.SemaphoreType.DMA,
            pltpu.SemaphoreType.DMA,
            pltpu.SemaphoreType.DMA,
        ],
    )(_sc_body)
    masked, tmpl = run(x.reshape(b, s, c), t_p.reshape(s * s))
    return (masked.reshape(b, h, w, c), x, tmpl.reshape(b, h, w, c))


# speedup vs baseline: 1.3315x; 1.0354x over previous
import functools

import jax
import jax.numpy as jnp
from jax import lax
from jax.experimental import pallas as pl
from jax.experimental.pallas import tpu as pltpu
from jax.experimental.pallas import tpu_sc as plsc

_H = 14
_W = 14
_S = _H * _W
_B = 64
_C = 768
_CG = 128
_L = 16
_NW = 32
_NCG = _C // _CG
_UNITS = _B * _NCG
_UPW = _UNITS // _NW


def _unit_pos(uid):
    return uid // _NCG, (uid % _NCG) * _CG


def _in_copies(x_hbm, b, c0, buf, sem):
    return [
        pltpu.make_async_copy(
            x_hbm.at[b, :, pl.ds(c0, _CG)], buf, sem)
    ]


def _buf_out(buf, dst_hbm, b, c0, sem):
    return [
        pltpu.make_async_copy(
            buf, dst_hbm.at[b, :, pl.ds(c0, _CG)], sem)
    ]


def _sc_body(x_hbm, tp_hbm, masked_hbm, tmpl_hbm, tp_v, x_a, x_b, t_v,
             sem_xa, sem_xb, sem_t, sem_ma, sem_mb):
    wid = lax.axis_index("s") * 2 + lax.axis_index("c")
    pltpu.sync_copy(tp_hbm, tp_v)

    def compute(x2):
        for g in range(_CG // _L):
            gl = g * _L

            mx0 = jnp.full((_L,), -jnp.inf, jnp.float32)
            am0 = jnp.zeros((_L,), jnp.int32)

            @plsc.parallel_loop(0, _S, 4, unroll=7, carry=(mx0, am0))
            def amax_loop(s, carry):
                mx, am = carry
                v0 = x2[s, pl.ds(gl, _L)]
                v1 = x2[s + 1, pl.ds(gl, _L)]
                v2 = x2[s + 2, pl.ds(gl, _L)]
                v3 = x2[s + 3, pl.ds(gl, _L)]
                b1 = v1 > v0
                m01 = jnp.where(b1, v1, v0)
                a01 = jnp.where(b1, s + 1, s)
                b3 = v3 > v2
                m23 = jnp.where(b3, v3, v2)
                a23 = jnp.where(b3, s + 3, s + 2)
                bb = m23 > m01
                ml = jnp.where(bb, m23, m01)
                al = jnp.where(bb, a23, a01)
                bc = ml > mx
                return (jnp.where(bc, ml, mx), jnp.where(bc, al, am))

            _, am = amax_loop
            base = am * _S

            @plsc.parallel_loop(0, _S, 2, unroll=7)
            def out_loop(s):
                for d in range(2):
                    tv = plsc.load_gather(tp_v, [base + (s + d)])
                    xv = x2[s + d, pl.ds(gl, _L)]
                    t_v[s + d, pl.ds(gl, _L)] = tv
                    x2[s + d, pl.ds(gl, _L)] = jnp.maximum(xv * tv, 0.0)

    b0, c00 = _unit_pos(wid)
    for sem in (sem_t, sem_mb):
        for cp in _in_copies(x_hbm, b0, c00, t_v, sem):
            cp.start()
    for cp in _in_copies(x_hbm, b0, c00, x_a, sem_xa):
        cp.start()

    def pair_body(j, _):
        u0 = (2 * j) * _NW + wid
        u1 = (2 * j + 1) * _NW + wid
        un = jnp.minimum(2 * j + 2, _UPW - 1) * _NW + wid
        bu0, cu0 = _unit_pos(u0)
        bu1, cu1 = _unit_pos(u1)
        bun, cun = _unit_pos(un)

        for cp in _buf_out(x_b, masked_hbm, bu0, cu0, sem_mb):
            cp.wait()
        for cp in _in_copies(x_hbm, bu1, cu1, x_b, sem_xb):
            cp.start()
        for cp in _in_copies(x_hbm, bu0, cu0, x_a, sem_xa):
            cp.wait()
        for cp in _buf_out(t_v, tmpl_hbm, bu0, cu0, sem_t):
            cp.wait()
        compute(x_a)
        for cp in _buf_out(t_v, tmpl_hbm, bu0, cu0, sem_t):
            cp.start()
        for cp in _buf_out(x_a, masked_hbm, bu0, cu0, sem_ma):
            cp.start()

        for cp in _buf_out(x_a, masked_hbm, bu1, cu1, sem_ma):
            cp.wait()
        for cp in _in_copies(x_hbm, bun, cun, x_a, sem_xa):
            cp.start()
        for cp in _in_copies(x_hbm, bu1, cu1, x_b, sem_xb):
            cp.wait()
        for cp in _buf_out(t_v, tmpl_hbm, bu1, cu1, sem_t):
            cp.wait()
        compute(x_b)
        for cp in _buf_out(t_v, tmpl_hbm, bu1, cu1, sem_t):
            cp.start()
        for cp in _buf_out(x_b, masked_hbm, bu1, cu1, sem_mb):
            cp.start()
        return 0

    lax.fori_loop(0, _UPW // 2, pair_body, 0)

    bl, cl = _unit_pos((_UPW - 1) * _NW + wid)
    for cp in _buf_out(t_v, tmpl_hbm, bl, cl, sem_t):
        cp.wait()
    for cp in _buf_out(x_b, masked_hbm, bl, cl, sem_mb):
        cp.wait()
    for cp in _in_copies(x_hbm, bl, cl, x_a, sem_xa):
        cp.wait()


def kernel(input, t_p):
    x = input
    b, h, w, c = x.shape

    mesh = plsc.VectorSubcoreMesh(core_axis_name="c", subcore_axis_name="s")
    s = h * w
    run = functools.partial(
        pl.kernel,
        out_type=[
            jax.ShapeDtypeStruct((b, s, c), jnp.float32),
            jax.ShapeDtypeStruct((b, s, c), jnp.float32),
        ],
        mesh=mesh,
        compiler_params=pltpu.CompilerParams(needs_layout_passes=False),
        scratch_types=[
            pltpu.VMEM((_S * _S,), jnp.float32),
            pltpu.VMEM((_S, _CG), jnp.float32),
            pltpu.VMEM((_S, _CG), jnp.float32),
            pltpu.VMEM((_S, _CG), jnp.float32),
            pltpu.SemaphoreType.DMA,
            pltpu.SemaphoreType.DMA,
            pltpu.SemaphoreType.DMA,
            pltpu.SemaphoreType.DMA,
            pltpu.SemaphoreType.DMA,
        ],
    )(_sc_body)
    masked, tmpl = run(x.reshape(b, s, c), t_p.reshape(s * s))
    return (masked.reshape(b, h, w, c), x, tmpl.reshape(b, h, w, c))
